# Initial kernel scaffold; baseline (speedup 1.0000x reference)
import jax
import jax.numpy as jnp
from jax.experimental import pallas as pl


def kernel(edges, node_features, W1, b1, W2, b2):
    raise NotImplementedError("write your pallas kernel here")


# speedup vs baseline: 7.4451x; 7.4451x over previous
import functools

import jax
import jax.numpy as jnp
from jax import lax
from jax.experimental import pallas as pl
from jax.experimental.pallas import tpu as pltpu
from jax.experimental.pallas import tpu_sc as plsc

N = 10000
E = 320000
D_IN = 128
D_HID = 512
D_OUT = 200

NC = 2
NS = 16
CH = 128
CHUNKS = -(-E // (NS * CH))
CHUNKS += CHUNKS % 2
E_PAD = NS * CHUNKS * CH
WIN = 5120
TRASH = WIN
W_PAD = 6144
RPW = W_PAD // NS

_MESH = plsc.VectorSubcoreMesh(
    core_axis_name="c", subcore_axis_name="s", num_cores=NC, num_subcores=NS
)


def _deg_body(dst_hbm, zeros_hbm, out_hbm, idx_v, ones_v, acc_sh, sem):
    c = lax.axis_index("c")
    s = lax.axis_index("s")
    for i in range(CH // 16):
        ones_v[pl.ds(i * 16, 16)] = jnp.full((16,), 1.0, jnp.float32)
    pltpu.sync_copy(zeros_hbm.at[pl.ds(s * RPW, RPW)], acc_sh.at[pl.ds(s * RPW, RPW)])
    pltpu.sync_copy(dst_hbm.at[c].at[s], idx_v)
    plsc.subcore_barrier()
    copies = [
        pltpu.async_copy(ones_v, acc_sh.at[idx_v.at[j]], sem, add=True)
        for j in range(CHUNKS)
    ]
    for cp in copies:
        cp.wait()
    plsc.subcore_barrier()
    pltpu.sync_copy(acc_sh.at[pl.ds(s * RPW, RPW)], out_hbm.at[c].at[pl.ds(s * RPW, RPW)])


@jax.jit
def _sc_degree(dst_idx, zeros1):
    k = pl.kernel(
        _deg_body,
        out_type=jax.ShapeDtypeStruct((NC, W_PAD), jnp.float32),
        mesh=_MESH,
        scratch_types=[
            pltpu.VMEM((CHUNKS, CH), jnp.int32),
            pltpu.VMEM((CH,), jnp.float32),
            pltpu.VMEM_SHARED((W_PAD,), jnp.float32),
            pltpu.SemaphoreType.DMA,
        ],
    )
    return k(dst_idx, zeros1)


NB = 2


def _agg_body(G, xg_hbm, src_hbm, dst_hbm, zeros_hbm, out_hbm,
              srcv, dstv, bufs, acc_sh, gsems, ssems):
    c = lax.axis_index("c")
    s = lax.axis_index("s")
    rows_mine = pl.ds(s * RPW, RPW)
    pltpu.sync_copy(src_hbm.at[s], srcv)
    pltpu.sync_copy(dst_hbm.at[c].at[s], dstv)
    for g in range(G):
        x_hbm = xg_hbm.at[g]
        pltpu.sync_copy(zeros_hbm.at[rows_mine], acc_sh.at[rows_mine])
        plsc.subcore_barrier()
        h_g = [None] * CHUNKS
        h_s = [None] * CHUNKS
        h_g[0] = pltpu.async_copy(x_hbm.at[srcv.at[0]], bufs[0], gsems[0])
        for j in range(CHUNKS):
            if j >= 1:
                h_s[j - 1].wait()
            if j + 1 < CHUNKS:
                h_g[j + 1] = pltpu.async_copy(
                    x_hbm.at[srcv.at[j + 1]], bufs[(j + 1) % NB], gsems[(j + 1) % 2]
                )
            h_g[j].wait()
            h_s[j] = pltpu.async_copy(
                bufs[j % NB], acc_sh.at[dstv.at[j]], ssems[j % 2], add=True
            )
        h_s[CHUNKS - 1].wait()
        plsc.subcore_barrier()
        pltpu.sync_copy(acc_sh.at[rows_mine], out_hbm.at[c].at[g].at[rows_mine])
        plsc.subcore_barrier()


@functools.partial(jax.jit, static_argnums=0)
def _sc_aggregate(G, xg, src_idx, dst_idx, zeros2):
    k = pl.kernel(
        functools.partial(_agg_body, G),
        out_type=jax.ShapeDtypeStruct((NC, G, W_PAD, D_IN), jnp.float32),
        mesh=_MESH,
        scratch_types=[
            pltpu.VMEM((CHUNKS, CH), jnp.int32),
            pltpu.VMEM((CHUNKS, CH), jnp.int32),
            [pltpu.VMEM((CH, D_IN), jnp.float32) for _ in range(NB)],
            pltpu.VMEM_SHARED((W_PAD, D_IN), jnp.float32),
            [pltpu.SemaphoreType.DMA for _ in range(2)],
            [pltpu.SemaphoreType.DMA for _ in range(2)],
        ],
    )
    return k(xg, src_idx, dst_idx, zeros2)


def _tc_body(raw0_ref, xs_ref, dis_ref, w1_ref, b1_ref, w2_ref, o_ref):
    dis = dis_ref[...]
    r = (raw0_ref[...] + xs_ref[...]) * dis
    h = jax.lax.dot(r, w1_ref[...], precision=jax.lax.Precision.HIGHEST)
    h = jnp.maximum(h + b1_ref[...], 0.0)
    t = jax.lax.dot(h, w2_ref[...], precision=jax.lax.Precision.HIGHEST)
    o_ref[...] = t * dis


BM = 1000


@jax.jit
def _tc_stage(raw0, xs, dis2, w1, b1r, w2):
    return pl.pallas_call(
        _tc_body,
        grid=(N // BM,),
        in_specs=[
            pl.BlockSpec((BM, D_IN), lambda i: (i, 0)),
            pl.BlockSpec((BM, D_IN), lambda i: (i, 0)),
            pl.BlockSpec((BM, 1), lambda i: (i, 0)),
            pl.BlockSpec((D_IN, D_HID), lambda i: (0, 0)),
            pl.BlockSpec((1, D_HID), lambda i: (0, 0)),
            pl.BlockSpec((D_HID, D_OUT), lambda i: (0, 0)),
        ],
        out_specs=pl.BlockSpec((BM, D_OUT), lambda i: (i, 0)),
        out_shape=jax.ShapeDtypeStruct((N, D_OUT), jnp.float32),
    )(raw0, xs, dis2, w1, b1r, w2)


def kernel(edges, node_features, W1, b1, W2, b2):
    pad = E_PAD - E
    src = jnp.concatenate([edges[0], jnp.zeros((pad,), jnp.int32)])
    dst = jnp.concatenate([edges[1], jnp.full((pad,), 3 * WIN, jnp.int32)])
    src_idx = src.reshape(NS, CHUNKS, CH)
    win = dst // WIN
    dst_remap = jnp.stack(
        [jnp.where(win == c, dst - c * WIN, TRASH) for c in range(NC)]
    ).reshape(NC, NS, CHUNKS, CH)

    zeros1 = jnp.zeros((W_PAD,), jnp.float32)
    zeros2 = jnp.zeros((W_PAD, D_IN), jnp.float32)

    degp = _sc_degree(dst_remap, zeros1)
    deg = jnp.concatenate([degp[0, :WIN], degp[1, :WIN]])[:N] + 1.0
    dis2 = lax.rsqrt(deg)[:, None]

    xs = node_features * dis2
    raw1 = _sc_aggregate(1, xs[None], src_idx, dst_remap, zeros2)
    raw1 = raw1[:, 0, :WIN].reshape(NC * WIN, D_IN)[:N]
    ts = _tc_stage(raw1, xs, dis2, W1, b1[None, :], W2)
    tsg = jnp.pad(ts, ((0, 0), (0, 2 * D_IN - D_OUT)))
    tsg = tsg.reshape(N, 2, D_IN).transpose(1, 0, 2)
    raw2 = _sc_aggregate(2, tsg, src_idx, dst_remap, zeros2)
    raw2 = raw2[:, :, :WIN].transpose(0, 2, 1, 3).reshape(NC * WIN, 2 * D_IN)
    raw2 = raw2[:N, :D_OUT]
    return dis2 * (raw2 + ts) + b2[None, :]
